# baseline (speedup 1.0000x reference)
import dataclasses

import jax
import jax.numpy as jnp
from jax import lax
from jax.experimental import pallas as pl
from jax.experimental.pallas import tpu as pltpu
from jax.experimental.pallas import tpu_sc as plsc

N = 10000
NE = 160000
D = 256
H = 128
DE = 65

NS = 16
CHUNK = 80
EPT = NE // NS
NCHUNK = EPT // CHUNK
RPT = 624
TAIL = N - NS * RPT

BLK_E = 4000
BLK_N = 1000


def _edge_linear_body(ea_ref, w_ref, b_ref, out_ref):
    ea = ea_ref[...]
    z = jnp.dot(ea.astype(jnp.bfloat16), w_ref[...].astype(jnp.bfloat16),
                preferred_element_type=jnp.float32) + b_ref[0]
    e = jnp.where(z > 0, z, jnp.exp(jnp.minimum(z, 0.0)) - 1.0)
    mask = (ea[:, 0:1] < 8.0).astype(jnp.float32)
    eb16 = (e * mask).astype(jnp.bfloat16)
    ei = jax.lax.bitcast_convert_type(eb16, jnp.uint16).astype(jnp.uint32)
    ei = ei.reshape(BLK_E // 2, 2, H)
    w = ei[:, 0, :] | (ei[:, 1, :] << 16)
    out_ref[...] = jax.lax.bitcast_convert_type(w, jnp.int32)


def _edge_linear(edge_attr, W_edge, b2):
    neb = NE // BLK_E
    return pl.pallas_call(
        _edge_linear_body,
        grid=(neb, 2),
        in_specs=[
            pl.BlockSpec((BLK_E, DE), lambda i, h: (i, 0)),
            pl.BlockSpec((DE, H), lambda i, h: (0, h)),
            pl.BlockSpec((1, 1, H), lambda i, h: (h, 0, 0)),
        ],
        out_specs=pl.BlockSpec((BLK_E // 2, H), lambda i, h: (h * neb + i, 0)),
        out_shape=jax.ShapeDtypeStruct((NE, H), jnp.int32),
    )(edge_attr, W_edge, b2)


def _sc_aggregate_body(edge_hbm, xh_hbm, src_hbm, tgt_hbm, zeros_hbm, out_hbm,
                       table, sidx2, tidx2, erows_a, erows_b, xrows2,
                       isem_s, isem_t, esem, gsem, ssem):
    erows = (erows_a, erows_b)
    c = lax.axis_index("c")
    s = lax.axis_index("s")

    pltpu.sync_copy(zeros_hbm, table.at[pl.ds(s * RPT, RPT)])

    @pl.when(s == 0)
    def _zero_tail():
        pltpu.sync_copy(zeros_hbm.at[pl.ds(0, TAIL)], table.at[pl.ds(NS * RPT, TAIL)])

    plsc.subcore_barrier()

    xoff = c * N
    tile_base = s * EPT

    def issue_idx(k, b):
        base = tile_base + k * CHUNK
        pltpu.make_async_copy(src_hbm.at[pl.ds(base, CHUNK)], sidx2.at[b],
                              isem_s.at[b]).start()
        pltpu.make_async_copy(tgt_hbm.at[pl.ds(base, CHUNK)], tidx2.at[b],
                              isem_t.at[b]).start()

    def wait_idx(k, b):
        base = tile_base + k * CHUNK
        pltpu.make_async_copy(src_hbm.at[pl.ds(base, CHUNK)], sidx2.at[b],
                              isem_s.at[b]).wait()
        pltpu.make_async_copy(tgt_hbm.at[pl.ds(base, CHUNK)], tidx2.at[b],
                              isem_t.at[b]).wait()

    def issue_edge(k, b):
        base = pl.multiple_of(c * (NE // 2) + s * (EPT // 2) + k * (CHUNK // 2), 8)
        pltpu.make_async_copy(edge_hbm.at[pl.ds(base, CHUNK // 2)], erows[b],
                              esem.at[b]).start()

    def wait_edge(k, b):
        base = pl.multiple_of(c * (NE // 2) + s * (EPT // 2) + k * (CHUNK // 2), 8)
        pltpu.make_async_copy(edge_hbm.at[pl.ds(base, CHUNK // 2)], erows[b],
                              esem.at[b]).wait()

    def gather_dma(b):
        return pltpu.make_async_copy(xh_hbm.at[tidx2.at[b]], xrows2.at[b],
                                     gsem.at[b])

    def scatter_dma(b):
        return pltpu.make_async_copy(xrows2.at[b], table.at[sidx2.at[b]],
                                     ssem.at[b])

    def chunk_body(k, b, first, next_issue):
        wait_idx(k, b)

        @pl.loop(0, CHUNK, step=16)
        def _off(j):
            tidx2[b, pl.ds(j, 16)] = tidx2[b, pl.ds(j, 16)] + xoff

        gather_dma(b).start()
        if next_issue:
            nb = 1 - b
            if not first:
                scatter_dma(nb).wait()
            issue_idx(k + 1, nb)
            issue_edge(k + 1, nb)
        wait_edge(k, b)
        gather_dma(b).wait()

        eb, xb = erows[b], xrows2.at[b]
        himask = jnp.full((16,), -65536, jnp.int32)

        @pl.loop(0, CHUNK // 2)
        def _mul(g):
            i0 = 2 * g
            for j in range(H // 16):
                sl = pl.ds(j * 16, 16)
                w = eb[g, sl]
                lo = plsc.bitcast(w << 16, jnp.float32)
                hi = plsc.bitcast(w & himask, jnp.float32)
                xb[i0, sl] = lo * xb[i0, sl]
                xb[i0 + 1, sl] = hi * xb[i0 + 1, sl]

        scatter_dma(b).start(add=True)

    issue_idx(0, 0)
    issue_edge(0, 0)
    chunk_body(0, 0, first=True, next_issue=True)

    @pl.loop(0, (NCHUNK - 1) // 2)
    def _pairs(p):
        chunk_body(1 + 2 * p, 1, first=False, next_issue=True)

        @pl.when(p < (NCHUNK - 1) // 2 - 1)
        def _more():
            chunk_body(2 + 2 * p, 0, first=False, next_issue=True)

        @pl.when(p == (NCHUNK - 1) // 2 - 1)
        def _last():
            chunk_body(2 + 2 * p, 0, first=False, next_issue=False)

    scatter_dma(1).wait()
    scatter_dma(0).wait()

    plsc.subcore_barrier()
    pltpu.sync_copy(table.at[pl.ds(s * RPT, RPT)],
                    out_hbm.at[pl.ds(c * N + s * RPT, RPT)])

    @pl.when(s == 0)
    def _out_tail():
        pltpu.sync_copy(table.at[pl.ds(NS * RPT, TAIL)],
                        out_hbm.at[pl.ds(c * N + NS * RPT, TAIL)])


def _sc_aggregate(edge2, xh, src, tgt, zeros):
    mesh = plsc.VectorSubcoreMesh(core_axis_name="c", subcore_axis_name="s")
    cp = pltpu.CompilerParams()
    if "needs_layout_passes" in pltpu.CompilerParams.__dataclass_fields__:
        cp = dataclasses.replace(cp, needs_layout_passes=False)
    kern = pl.kernel(
        _sc_aggregate_body,
        out_type=jax.ShapeDtypeStruct((2 * N, H), jnp.float32),
        compiler_params=cp,
        mesh=mesh,
        scratch_types=[
            pltpu.VMEM_SHARED((N, H), jnp.float32),
            pltpu.VMEM((2, CHUNK), jnp.int32),
            pltpu.VMEM((2, CHUNK), jnp.int32),
            pltpu.VMEM((CHUNK // 2, H), jnp.int32),
            pltpu.VMEM((CHUNK // 2, H), jnp.int32),
            pltpu.VMEM((2, CHUNK, H), jnp.float32),
            pltpu.SemaphoreType.DMA((2,)),
            pltpu.SemaphoreType.DMA((2,)),
            pltpu.SemaphoreType.DMA((2,)),
            pltpu.SemaphoreType.DMA((2,)),
            pltpu.SemaphoreType.DMA((2,)),
        ],
    )
    return kern(edge2, xh, src, tgt, zeros)


def _update_body(x_ref, alo_ref, ahi_ref, w_ref, b_ref, out_ref):
    a = jnp.concatenate([alo_ref[...], ahi_ref[...]], axis=1)
    h = x_ref[...] + a
    out_ref[...] = jnp.dot(h, w_ref[...], preferred_element_type=jnp.float32) + b_ref[...]


def _update(x, aggr2, W_upd, bu2):
    nnb = N // BLK_N
    return pl.pallas_call(
        _update_body,
        grid=(nnb,),
        in_specs=[
            pl.BlockSpec((BLK_N, D), lambda i: (i, 0)),
            pl.BlockSpec((BLK_N, H), lambda i: (i, 0)),
            pl.BlockSpec((BLK_N, H), lambda i: (nnb + i, 0)),
            pl.BlockSpec((D, D), lambda i: (0, 0)),
            pl.BlockSpec((1, D), lambda i: (0, 0)),
        ],
        out_specs=pl.BlockSpec((BLK_N, D), lambda i: (i, 0)),
        out_shape=jax.ShapeDtypeStruct((N, D), jnp.float32),
    )(x, aggr2, aggr2, W_upd, bu2)


def kernel(x, edge_index, edge_attr, W_edge, b_edge, W_upd, b_upd):
    src = edge_index[0]
    tgt = edge_index[1]
    xh = x.reshape(N, 2, H).swapaxes(0, 1).reshape(2 * N, H)
    b2 = b_edge.reshape(2, 1, H)
    bu2 = b_upd.reshape(1, D)
    zeros = jnp.zeros((RPT, H), jnp.float32)

    edge2 = _edge_linear(edge_attr, W_edge, b2)
    aggr2 = _sc_aggregate(edge2, xh, src, tgt, zeros)
    return _update(x, aggr2, W_upd, bu2)


# speedup vs baseline: 1.5238x; 1.5238x over previous
import functools

import jax
import jax.numpy as jnp
from jax import lax
from jax.experimental import pallas as pl
from jax.experimental.pallas import tpu as pltpu
from jax.experimental.pallas import tpu_sc as plsc

N = 10000
NE = 160000
D = 256
H = 128
DE = 65

NS = 16
CHUNK = 80
EPT = NE // NS
NCHUNK = EPT // CHUNK
RPT = 624
TAIL = N - NS * RPT

BLK_E = 4000
BLK_N = 1000


def _edge_linear_body(ea_ref, w_ref, b_ref, out_ref):
    ea = ea_ref[...]
    z = jnp.dot(ea, w_ref[...], preferred_element_type=jnp.float32) + b_ref[0]
    e = jnp.where(z > 0, z, jnp.exp(jnp.minimum(z, 0.0)) - 1.0)
    mask = (ea[:, 0:1] < 8.0).astype(jnp.float32)
    out_ref[...] = e * mask


def _edge_linear(edge_attr, W_edge, b2, e0, ne):
    neb = ne // BLK_E
    ob = e0 // BLK_E
    return pl.pallas_call(
        _edge_linear_body,
        grid=(neb, 2),
        in_specs=[
            pl.BlockSpec((BLK_E, DE), lambda i, h: (ob + i, 0)),
            pl.BlockSpec((DE, H), lambda i, h: (0, h)),
            pl.BlockSpec((1, 1, H), lambda i, h: (h, 0, 0)),
        ],
        out_specs=pl.BlockSpec((BLK_E, H), lambda i, h: (h * neb + i, 0)),
        out_shape=jax.ShapeDtypeStruct((2 * ne, H), jnp.float32),
    )(edge_attr, W_edge, b2)


def _sc_aggregate_body(ebase, ne, edge_hbm, xh_hbm, src_hbm, tgt_hbm,
                       zeros_hbm, out_hbm,
                       table, sidx2, tidx2, erows2, xrows2,
                       isem_s, isem_t, esem, gsem, ssem):
    ept = ne // NS
    nchunk = ept // CHUNK
    c = lax.axis_index("c")
    s = lax.axis_index("s")

    pltpu.sync_copy(zeros_hbm, table.at[pl.ds(s * RPT, RPT)])

    @pl.when(s == 0)
    def _zero_tail():
        pltpu.sync_copy(zeros_hbm.at[pl.ds(0, TAIL)], table.at[pl.ds(NS * RPT, TAIL)])

    plsc.subcore_barrier()

    xoff = c * N
    tile_base = ebase + s * ept

    def issue_idx(k, b):
        base = tile_base + k * CHUNK
        pltpu.make_async_copy(src_hbm.at[pl.ds(base, CHUNK)], sidx2.at[b],
                              isem_s.at[b]).start()
        pltpu.make_async_copy(tgt_hbm.at[pl.ds(base, CHUNK)], tidx2.at[b],
                              isem_t.at[b]).start()

    def wait_idx(k, b):
        base = tile_base + k * CHUNK
        pltpu.make_async_copy(src_hbm.at[pl.ds(base, CHUNK)], sidx2.at[b],
                              isem_s.at[b]).wait()
        pltpu.make_async_copy(tgt_hbm.at[pl.ds(base, CHUNK)], tidx2.at[b],
                              isem_t.at[b]).wait()

    def issue_edge(k, b):
        base = c * ne + s * ept + k * CHUNK
        pltpu.make_async_copy(edge_hbm.at[pl.ds(base, CHUNK)], erows2.at[b],
                              esem.at[b]).start()

    def wait_edge(k, b):
        base = c * ne + s * ept + k * CHUNK
        pltpu.make_async_copy(edge_hbm.at[pl.ds(base, CHUNK)], erows2.at[b],
                              esem.at[b]).wait()

    def gather_dma(b):
        return pltpu.make_async_copy(xh_hbm.at[tidx2.at[b]], xrows2.at[b],
                                     gsem.at[b])

    def scatter_dma(b):
        return pltpu.make_async_copy(erows2.at[b], table.at[sidx2.at[b]],
                                     ssem.at[b])

    def chunk_body(k, b, first, next_issue):
        wait_idx(k, b)

        @pl.loop(0, CHUNK, step=16)
        def _off(j):
            tidx2[b, pl.ds(j, 16)] = tidx2[b, pl.ds(j, 16)] + xoff

        gather_dma(b).start()
        if next_issue:
            nb = 1 - b
            if not first:
                scatter_dma(nb).wait()
            issue_idx(k + 1, nb)
            issue_edge(k + 1, nb)
        wait_edge(k, b)
        gather_dma(b).wait()

        eb, xb = erows2.at[b], xrows2.at[b]

        @pl.loop(0, CHUNK)
        def _mul(i):
            for j in range(H // 16):
                sl = pl.ds(j * 16, 16)
                eb[i, sl] = eb[i, sl] * xb[i, sl]

        scatter_dma(b).start(add=True)

    issue_idx(0, 0)
    issue_edge(0, 0)
    chunk_body(0, 0, first=True, next_issue=True)

    npair = (nchunk - 1) // 2
    odd_tail = (nchunk - 1) % 2 == 1

    @pl.loop(0, npair)
    def _pairs(p):
        chunk_body(1 + 2 * p, 1, first=False, next_issue=True)

        @pl.when(odd_tail | (p < npair - 1))
        def _more():
            chunk_body(2 + 2 * p, 0, first=False, next_issue=True)

        @pl.when((not odd_tail) & (p == npair - 1))
        def _last():
            chunk_body(2 + 2 * p, 0, first=False, next_issue=False)

    if odd_tail:
        chunk_body(nchunk - 1, (nchunk - 1) % 2, first=False,
                   next_issue=False)

    scatter_dma(1).wait()
    scatter_dma(0).wait()

    plsc.subcore_barrier()
    pltpu.sync_copy(table.at[pl.ds(s * RPT, RPT)],
                    out_hbm.at[pl.ds(c * N + s * RPT, RPT)])

    @pl.when(s == 0)
    def _out_tail():
        pltpu.sync_copy(table.at[pl.ds(NS * RPT, TAIL)],
                        out_hbm.at[pl.ds(c * N + NS * RPT, TAIL)])


def _sc_aggregate(edge2, xh, src, tgt, zeros, ebase, ne):
    import functools as _ft
    mesh = plsc.VectorSubcoreMesh(core_axis_name="c", subcore_axis_name="s")
    kern = pl.kernel(
        _ft.partial(_sc_aggregate_body, ebase, ne),
        out_type=jax.ShapeDtypeStruct((2 * N, H), jnp.float32),
        mesh=mesh,
        scratch_types=[
            pltpu.VMEM_SHARED((N, H), jnp.float32),
            pltpu.VMEM((2, CHUNK), jnp.int32),
            pltpu.VMEM((2, CHUNK), jnp.int32),
            pltpu.VMEM((2, CHUNK, H), jnp.float32),
            pltpu.VMEM((2, CHUNK, H), jnp.float32),
            pltpu.SemaphoreType.DMA((2,)),
            pltpu.SemaphoreType.DMA((2,)),
            pltpu.SemaphoreType.DMA((2,)),
            pltpu.SemaphoreType.DMA((2,)),
            pltpu.SemaphoreType.DMA((2,)),
        ],
    )
    return kern(edge2, xh, src, tgt, zeros)


def _update_body(x_ref, alo_a, ahi_a, alo_b, ahi_b, w_ref, b_ref, out_ref):
    a = jnp.concatenate([alo_a[...] + alo_b[...], ahi_a[...] + ahi_b[...]],
                        axis=1)
    h = x_ref[...] + a
    out_ref[...] = jnp.dot(h, w_ref[...], preferred_element_type=jnp.float32) + b_ref[...]


def _update(x, aggr_a, aggr_b, W_upd, bu2):
    nnb = N // BLK_N
    return pl.pallas_call(
        _update_body,
        grid=(nnb,),
        in_specs=[
            pl.BlockSpec((BLK_N, D), lambda i: (i, 0)),
            pl.BlockSpec((BLK_N, H), lambda i: (i, 0)),
            pl.BlockSpec((BLK_N, H), lambda i: (nnb + i, 0)),
            pl.BlockSpec((BLK_N, H), lambda i: (i, 0)),
            pl.BlockSpec((BLK_N, H), lambda i: (nnb + i, 0)),
            pl.BlockSpec((D, D), lambda i: (0, 0)),
            pl.BlockSpec((1, D), lambda i: (0, 0)),
        ],
        out_specs=pl.BlockSpec((BLK_N, D), lambda i: (i, 0)),
        out_shape=jax.ShapeDtypeStruct((N, D), jnp.float32),
    )(x, aggr_a, aggr_a, aggr_b, aggr_b, W_upd, bu2)


def kernel(x, edge_index, edge_attr, W_edge, b_edge, W_upd, b_upd):
    src = edge_index[0]
    tgt = edge_index[1]
    xh = x.reshape(N, 2, H).swapaxes(0, 1).reshape(2 * N, H)
    b2 = b_edge.reshape(2, 1, H)
    bu2 = b_upd.reshape(1, D)
    zeros = jnp.zeros((RPT, H), jnp.float32)

    NE_A, NE_B = 64000, 96000
    edge_a = _edge_linear(edge_attr, W_edge, b2, 0, NE_A)
    edge_b = _edge_linear(edge_attr, W_edge, b2, NE_A, NE_B)
    aggr_a = _sc_aggregate(edge_a, xh, src, tgt, zeros, 0, NE_A)
    aggr_b = _sc_aggregate(edge_b, xh, src, tgt, zeros, NE_A, NE_B)
    return _update(x, aggr_a, aggr_b, W_upd, bu2)
